# Initial kernel scaffold; baseline (speedup 1.0000x reference)
import jax
import jax.numpy as jnp
from jax.experimental import pallas as pl


def kernel(x, edge_index, edge_weight, numNode, W_lin, b_lin, W_conv, att_src, att_dst, b_conv):
    raise NotImplementedError("write your pallas kernel here")


# speedup vs baseline: 18.5752x; 18.5752x over previous
import dataclasses
import functools

import jax
import jax.numpy as jnp
from jax import lax
from jax.experimental import pallas as pl
from jax.experimental.pallas import tpu as pltpu
from jax.experimental.pallas import tpu_sc as plsc

N = 10000
E = 320000
H = 128
NEG = 0.2
NUM_TILES = 32
EPT = E // NUM_TILES
CW = 128
CHUNKS = 80
EPT_PAD = CHUNKS * CW
NP = 10240
ROWS_PER_SUB = NP // 16
R = 1000


def _first_tc(x, wl, bl, wc, asw, adw):
    def body(x_ref, wl_ref, bl_ref, wc_ref, asw_ref, adw_ref,
             h_ref, h2_ref, as_ref, ad_ref, c_ref):
        xb = x_ref[...]
        hb = jnp.dot(xb, wl_ref[...], preferred_element_type=jnp.float32)
        hb = hb + bl_ref[...]
        h_ref[...] = hb
        h2 = jnp.dot(hb, wc_ref[...], preferred_element_type=jnp.float32)
        h2_ref[...] = h2
        a_s = jnp.sum(h2 * asw_ref[...], axis=1, keepdims=True)
        a_d = jnp.sum(h2 * adw_ref[...], axis=1, keepdims=True)
        as_ref[...] = a_s
        ad_ref[...] = a_d
        bm = jnp.max(a_s)
        i = pl.program_id(0)

        @pl.when(i == 0)
        def _():
            c_ref[0, 0] = bm

        @pl.when(i > 0)
        def _():
            c_ref[0, 0] = jnp.maximum(c_ref[0, 0], bm)

    return pl.pallas_call(
        body,
        grid=(N // R,),
        in_specs=[
            pl.BlockSpec((R, H), lambda i: (i, 0)),
            pl.BlockSpec((H, H), lambda i: (0, 0)),
            pl.BlockSpec((1, H), lambda i: (0, 0)),
            pl.BlockSpec((H, H), lambda i: (0, 0)),
            pl.BlockSpec((1, H), lambda i: (0, 0)),
            pl.BlockSpec((1, H), lambda i: (0, 0)),
        ],
        out_specs=[
            pl.BlockSpec((R, H), lambda i: (i, 0)),
            pl.BlockSpec((R, H), lambda i: (i, 0)),
            pl.BlockSpec((R, 1), lambda i: (i, 0)),
            pl.BlockSpec((R, 1), lambda i: (i, 0)),
            pl.BlockSpec(memory_space=pltpu.SMEM),
        ],
        out_shape=[
            jax.ShapeDtypeStruct((N, H), jnp.float32),
            jax.ShapeDtypeStruct((N, H), jnp.float32),
            jax.ShapeDtypeStruct((N, 1), jnp.float32),
            jax.ShapeDtypeStruct((N, 1), jnp.float32),
            jax.ShapeDtypeStruct((1, 1), jnp.float32),
        ],
    )(x, wl, bl, wc, asw, adw)


def _mid_tc(h_prev, r_in, num, den, bc, wc, asw, adw, nn, first):
    def body(h_ref, r_ref, num_ref, den_ref, bc_ref, wc_ref, asw_ref,
             adw_ref, nn_ref, h_out, r_out, h2_out, as_out, ad_out, c_out):
        i = pl.program_id(0)
        nb = num_ref[...]
        db = den_ref[...]
        t = (nb[0] + nb[1]) / (db[0] + db[1] + 1e-16) + bc_ref[...]
        hn = h_ref[...] + t
        h_out[...] = hn
        rows = i * R + lax.broadcasted_iota(jnp.int32, (R, 1), 0)
        mask = rows < nn_ref[0]
        hm = jnp.where(mask, hn, 0.0)
        if first:
            r_out[...] = hm
        else:
            r_out[...] = jnp.maximum(r_ref[...], hm)
        h2 = jnp.dot(hn, wc_ref[...], preferred_element_type=jnp.float32)
        h2_out[...] = h2
        a_s = jnp.sum(h2 * asw_ref[...], axis=1, keepdims=True)
        a_d = jnp.sum(h2 * adw_ref[...], axis=1, keepdims=True)
        as_out[...] = a_s
        ad_out[...] = a_d
        bm = jnp.max(a_s)

        @pl.when(i == 0)
        def _():
            c_out[0, 0] = bm

        @pl.when(i > 0)
        def _():
            c_out[0, 0] = jnp.maximum(c_out[0, 0], bm)

    return pl.pallas_call(
        body,
        grid=(N // R,),
        in_specs=[
            pl.BlockSpec((R, H), lambda i: (i, 0)),
            pl.BlockSpec((R, H), lambda i: (i, 0)),
            pl.BlockSpec((2, R, H), lambda i: (0, i, 0)),
            pl.BlockSpec((2, R, 1), lambda i: (0, i, 0)),
            pl.BlockSpec((1, H), lambda i: (0, 0)),
            pl.BlockSpec((H, H), lambda i: (0, 0)),
            pl.BlockSpec((1, H), lambda i: (0, 0)),
            pl.BlockSpec((1, H), lambda i: (0, 0)),
            pl.BlockSpec(memory_space=pltpu.SMEM),
        ],
        out_specs=[
            pl.BlockSpec((R, H), lambda i: (i, 0)),
            pl.BlockSpec((R, H), lambda i: (i, 0)),
            pl.BlockSpec((R, H), lambda i: (i, 0)),
            pl.BlockSpec((R, 1), lambda i: (i, 0)),
            pl.BlockSpec((R, 1), lambda i: (i, 0)),
            pl.BlockSpec(memory_space=pltpu.SMEM),
        ],
        out_shape=[
            jax.ShapeDtypeStruct((N, H), jnp.float32),
            jax.ShapeDtypeStruct((N, H), jnp.float32),
            jax.ShapeDtypeStruct((N, H), jnp.float32),
            jax.ShapeDtypeStruct((N, 1), jnp.float32),
            jax.ShapeDtypeStruct((N, 1), jnp.float32),
            jax.ShapeDtypeStruct((1, 1), jnp.float32),
        ],
    )(h_prev, r_in, num, den, bc, wc, asw, adw, nn)


def _final_tc(r_in, num, den, bc, nn):
    def body(r_ref, num_ref, den_ref, bc_ref, nn_ref, ret_out):
        i = pl.program_id(0)
        nb = num_ref[...]
        db = den_ref[...]
        t = (nb[0] + nb[1]) / (db[0] + db[1] + 1e-16) + bc_ref[...]
        rows = i * R + lax.broadcasted_iota(jnp.int32, (R, 1), 0)
        mask = rows < nn_ref[0]
        tm = jnp.where(mask, t, 0.0)
        ret_out[...] = jnp.maximum(r_ref[...], tm)

    return pl.pallas_call(
        body,
        grid=(N // R,),
        in_specs=[
            pl.BlockSpec((R, H), lambda i: (i, 0)),
            pl.BlockSpec((2, R, H), lambda i: (0, i, 0)),
            pl.BlockSpec((2, R, 1), lambda i: (0, i, 0)),
            pl.BlockSpec((1, H), lambda i: (0, 0)),
            pl.BlockSpec(memory_space=pltpu.SMEM),
        ],
        out_specs=[pl.BlockSpec((R, H), lambda i: (i, 0))],
        out_shape=[jax.ShapeDtypeStruct((N, H), jnp.float32)],
    )(r_in, num, den, bc, nn)[0]


def _sc_edge(h2, asv, adv, cvec, src_t, dst_t, w_t):
    mesh = plsc.VectorSubcoreMesh(core_axis_name="c", subcore_axis_name="s")
    cp = pltpu.CompilerParams()
    if "needs_layout_passes" in pltpu.CompilerParams.__dataclass_fields__:
        cp = dataclasses.replace(cp, needs_layout_passes=False)
    out_types = (
        jax.ShapeDtypeStruct((2, NP, H), jnp.float32),
        jax.ShapeDtypeStruct((2, NP), jnp.float32),
    )

    @functools.partial(
        pl.kernel,
        out_type=out_types,
        mesh=mesh,
        compiler_params=cp,
        scratch_types=[
            pltpu.VMEM_SHARED((NP, H), jnp.float32),
            pltpu.VMEM_SHARED((NP,), jnp.float32),
        ],
    )
    def k(h2_hbm, as_hbm, ad_hbm, c_hbm, src_hbm, dst_hbm, w_hbm,
          num_out, den_out, num_sh, den_sh):
        cid = lax.axis_index("c")
        sid = lax.axis_index("s")
        tid = cid * 16 + sid

        def run(as_tab, ad_tab, c_tab, src_blk, dst_blk, w_blk, vbuf,
                pbuf, rows, zbuf):
            pltpu.sync_copy(as_hbm, as_tab)
            pltpu.sync_copy(ad_hbm, ad_tab)
            pltpu.sync_copy(c_hbm, c_tab)

            zv = jnp.zeros((16,), jnp.float32)

            @pl.loop(0, CW)
            def _(r):
                for k8 in range(8):
                    rows[r, pl.ds(k8 * 16, 16)] = zv

            @pl.loop(0, ROWS_PER_SUB // 16)
            def _(g):
                zbuf[pl.ds(g * 16, 16)] = zv

            base = sid * ROWS_PER_SUB
            for kk in range(5):
                pltpu.sync_copy(rows.at[pl.ds(0, CW)],
                                num_sh.at[pl.ds(base + kk * CW, CW)])

            pltpu.sync_copy(zbuf, den_sh.at[pl.ds(base, ROWS_PER_SUB)])

            plsc.subcore_barrier()

            c_vec = c_tab[...]

            @pl.loop(0, CHUNKS // 8)
            def _(blk):
                pltpu.sync_copy(src_hbm.at[tid, pl.ds(blk * 8, 8)], src_blk)
                pltpu.sync_copy(dst_hbm.at[tid, pl.ds(blk * 8, 8)], dst_blk)
                pltpu.sync_copy(w_hbm.at[tid, pl.ds(blk * 8, 8)], w_blk)

                for jb in range(8):
                    for g in range(8):
                        sl = pl.ds(g * 16, 16)
                        src_v = src_blk[jb, sl]
                        dst_v = dst_blk[jb, sl]
                        as_v = plsc.load_gather(as_tab, [src_v])
                        ad_v = plsc.load_gather(ad_tab, [dst_v])
                        e = as_v + ad_v
                        e = jnp.maximum(e, NEG * e)
                        bb = c_vec + ad_v
                        bb = jnp.maximum(bb, NEG * bb)
                        p = jnp.exp(e - bb)
                        valid = jnp.where(
                            (blk * 8 + jb) * CW + g * 16 < EPT,
                            jnp.float32(1.0), jnp.float32(0.0))
                        p = p * valid
                        vbuf[sl] = p * w_blk[jb, sl]
                        pbuf[sl] = p

                    pltpu.sync_copy(h2_hbm.at[src_blk.at[jb]], rows)

                    @pl.loop(0, CW)
                    def _(r):
                        idxv = jnp.full((16,), r, jnp.int32)
                        sv = plsc.load_gather(vbuf, [idxv])
                        for k8 in range(8):
                            slk = pl.ds(k8 * 16, 16)
                            rows[r, slk] = rows[r, slk] * sv

                    pltpu.sync_copy(rows, num_sh.at[dst_blk.at[jb]],
                                    add=True)
                    pltpu.sync_copy(pbuf, den_sh.at[dst_blk.at[jb]],
                                    add=True)

            plsc.subcore_barrier()

            pltpu.sync_copy(num_sh.at[pl.ds(base, ROWS_PER_SUB)],
                            num_out.at[cid, pl.ds(base, ROWS_PER_SUB)])
            pltpu.sync_copy(den_sh.at[pl.ds(base, ROWS_PER_SUB)],
                            den_out.at[cid, pl.ds(base, ROWS_PER_SUB)])

        pl.run_scoped(
            run,
            as_tab=pltpu.VMEM((N,), jnp.float32),
            ad_tab=pltpu.VMEM((N,), jnp.float32),
            c_tab=pltpu.VMEM((16,), jnp.float32),
            src_blk=pltpu.VMEM((8, CW), jnp.int32),
            dst_blk=pltpu.VMEM((8, CW), jnp.int32),
            w_blk=pltpu.VMEM((8, CW), jnp.float32),
            vbuf=pltpu.VMEM((CW,), jnp.float32),
            pbuf=pltpu.VMEM((CW,), jnp.float32),
            rows=pltpu.VMEM((CW, H), jnp.float32),
            zbuf=pltpu.VMEM((ROWS_PER_SUB,), jnp.float32),
        )

    return k(h2, asv, adv, cvec, src_t, dst_t, w_t)


def kernel(x, edge_index, edge_weight, numNode, W_lin, b_lin, W_conv,
           att_src, att_dst, b_conv):
    src = edge_index[0]
    dst = edge_index[1]
    pad = EPT_PAD - EPT
    src_t = jnp.pad(src.reshape(NUM_TILES, EPT), ((0, 0), (0, pad)))
    src_t = src_t.reshape(NUM_TILES, CHUNKS, CW)
    dst_t = jnp.pad(dst.reshape(NUM_TILES, EPT), ((0, 0), (0, pad)))
    dst_t = dst_t.reshape(NUM_TILES, CHUNKS, CW)
    w_t = jnp.pad(edge_weight.reshape(NUM_TILES, EPT), ((0, 0), (0, pad)))
    w_t = w_t.reshape(NUM_TILES, CHUNKS, CW)
    nn = jnp.asarray(numNode, jnp.int32).reshape(1)

    h, h2c, asv, adv, c = _first_tc(
        x, W_lin, b_lin.reshape(1, H), W_conv[0],
        att_src[0].reshape(1, H), att_dst[0].reshape(1, H))

    r = h
    for i in range(3):
        cvec = jnp.broadcast_to(c.reshape(()), (16,))
        num, den = _sc_edge(h2c, asv.reshape(N), adv.reshape(N), cvec,
                            src_t, dst_t, w_t)
        den = den.reshape(2, NP, 1)
        bc = b_conv[i].reshape(1, H)
        if i < 2:
            h, r, h2c, asv, adv, c = _mid_tc(
                h, r, num, den, bc, W_conv[i + 1],
                att_src[i + 1].reshape(1, H), att_dst[i + 1].reshape(1, H),
                nn, first=(i == 0))
        else:
            return _final_tc(r, num, den, bc, nn)


# speedup vs baseline: 20.6870x; 1.1137x over previous
import dataclasses
import functools

import jax
import jax.numpy as jnp
from jax import lax
from jax.experimental import pallas as pl
from jax.experimental.pallas import tpu as pltpu
from jax.experimental.pallas import tpu_sc as plsc

N = 10000
E = 320000
H = 128
NEG = 0.2
NUM_TILES = 32
EPT = E // NUM_TILES
CW = 128
CHUNKS = 80
EPT_PAD = CHUNKS * CW
NP = 10240
ROWS_PER_SUB = NP // 16
R = 1000


def _first_tc(x, wl, bl, wc, asw, adw):
    def body(x_ref, wl_ref, bl_ref, wc_ref, asw_ref, adw_ref,
             h_ref, h2_ref, as_ref, ad_ref, c_ref):
        xb = x_ref[...]
        hb = jnp.dot(xb, wl_ref[...], preferred_element_type=jnp.float32)
        hb = hb + bl_ref[...]
        h_ref[...] = hb
        h2 = jnp.dot(hb, wc_ref[...], preferred_element_type=jnp.float32)
        h2_ref[...] = h2
        a_s = jnp.sum(h2 * asw_ref[...], axis=1, keepdims=True)
        a_d = jnp.sum(h2 * adw_ref[...], axis=1, keepdims=True)
        as_ref[...] = a_s
        ad_ref[...] = a_d
        bm = jnp.max(a_s)
        i = pl.program_id(0)

        @pl.when(i == 0)
        def _():
            c_ref[0, 0] = bm

        @pl.when(i > 0)
        def _():
            c_ref[0, 0] = jnp.maximum(c_ref[0, 0], bm)

    return pl.pallas_call(
        body,
        grid=(N // R,),
        in_specs=[
            pl.BlockSpec((R, H), lambda i: (i, 0)),
            pl.BlockSpec((H, H), lambda i: (0, 0)),
            pl.BlockSpec((1, H), lambda i: (0, 0)),
            pl.BlockSpec((H, H), lambda i: (0, 0)),
            pl.BlockSpec((1, H), lambda i: (0, 0)),
            pl.BlockSpec((1, H), lambda i: (0, 0)),
        ],
        out_specs=[
            pl.BlockSpec((R, H), lambda i: (i, 0)),
            pl.BlockSpec((R, H), lambda i: (i, 0)),
            pl.BlockSpec((R, 1), lambda i: (i, 0)),
            pl.BlockSpec((R, 1), lambda i: (i, 0)),
            pl.BlockSpec(memory_space=pltpu.SMEM),
        ],
        out_shape=[
            jax.ShapeDtypeStruct((N, H), jnp.float32),
            jax.ShapeDtypeStruct((N, H), jnp.float32),
            jax.ShapeDtypeStruct((N, 1), jnp.float32),
            jax.ShapeDtypeStruct((N, 1), jnp.float32),
            jax.ShapeDtypeStruct((1, 1), jnp.float32),
        ],
    )(x, wl, bl, wc, asw, adw)


def _mid_tc(h_prev, r_in, num, den, bc, wc, asw, adw, nn, first):
    def body(h_ref, r_ref, num_ref, den_ref, bc_ref, wc_ref, asw_ref,
             adw_ref, nn_ref, h_out, r_out, h2_out, as_out, ad_out, c_out):
        i = pl.program_id(0)
        nb = num_ref[...]
        db = den_ref[...]
        t = (nb[0] + nb[1]) / (db[0] + db[1] + 1e-16) + bc_ref[...]
        hn = h_ref[...] + t
        h_out[...] = hn
        rows = i * R + lax.broadcasted_iota(jnp.int32, (R, 1), 0)
        mask = rows < nn_ref[0]
        hm = jnp.where(mask, hn, 0.0)
        if first:
            r_out[...] = hm
        else:
            r_out[...] = jnp.maximum(r_ref[...], hm)
        h2 = jnp.dot(hn, wc_ref[...], preferred_element_type=jnp.float32)
        h2_out[...] = h2
        a_s = jnp.sum(h2 * asw_ref[...], axis=1, keepdims=True)
        a_d = jnp.sum(h2 * adw_ref[...], axis=1, keepdims=True)
        as_out[...] = a_s
        ad_out[...] = a_d
        bm = jnp.max(a_s)

        @pl.when(i == 0)
        def _():
            c_out[0, 0] = bm

        @pl.when(i > 0)
        def _():
            c_out[0, 0] = jnp.maximum(c_out[0, 0], bm)

    return pl.pallas_call(
        body,
        grid=(N // R,),
        in_specs=[
            pl.BlockSpec((R, H), lambda i: (i, 0)),
            pl.BlockSpec((R, H), lambda i: (i, 0)),
            pl.BlockSpec((2, R, H), lambda i: (0, i, 0)),
            pl.BlockSpec((2, R, 1), lambda i: (0, i, 0)),
            pl.BlockSpec((1, H), lambda i: (0, 0)),
            pl.BlockSpec((H, H), lambda i: (0, 0)),
            pl.BlockSpec((1, H), lambda i: (0, 0)),
            pl.BlockSpec((1, H), lambda i: (0, 0)),
            pl.BlockSpec(memory_space=pltpu.SMEM),
        ],
        out_specs=[
            pl.BlockSpec((R, H), lambda i: (i, 0)),
            pl.BlockSpec((R, H), lambda i: (i, 0)),
            pl.BlockSpec((R, H), lambda i: (i, 0)),
            pl.BlockSpec((R, 1), lambda i: (i, 0)),
            pl.BlockSpec((R, 1), lambda i: (i, 0)),
            pl.BlockSpec(memory_space=pltpu.SMEM),
        ],
        out_shape=[
            jax.ShapeDtypeStruct((N, H), jnp.float32),
            jax.ShapeDtypeStruct((N, H), jnp.float32),
            jax.ShapeDtypeStruct((N, H), jnp.float32),
            jax.ShapeDtypeStruct((N, 1), jnp.float32),
            jax.ShapeDtypeStruct((N, 1), jnp.float32),
            jax.ShapeDtypeStruct((1, 1), jnp.float32),
        ],
    )(h_prev, r_in, num, den, bc, wc, asw, adw, nn)


def _final_tc(r_in, num, den, bc, nn):
    def body(r_ref, num_ref, den_ref, bc_ref, nn_ref, ret_out):
        i = pl.program_id(0)
        nb = num_ref[...]
        db = den_ref[...]
        t = (nb[0] + nb[1]) / (db[0] + db[1] + 1e-16) + bc_ref[...]
        rows = i * R + lax.broadcasted_iota(jnp.int32, (R, 1), 0)
        mask = rows < nn_ref[0]
        tm = jnp.where(mask, t, 0.0)
        ret_out[...] = jnp.maximum(r_ref[...], tm)

    return pl.pallas_call(
        body,
        grid=(N // R,),
        in_specs=[
            pl.BlockSpec((R, H), lambda i: (i, 0)),
            pl.BlockSpec((2, R, H), lambda i: (0, i, 0)),
            pl.BlockSpec((2, R, 1), lambda i: (0, i, 0)),
            pl.BlockSpec((1, H), lambda i: (0, 0)),
            pl.BlockSpec(memory_space=pltpu.SMEM),
        ],
        out_specs=[pl.BlockSpec((R, H), lambda i: (i, 0))],
        out_shape=[jax.ShapeDtypeStruct((N, H), jnp.float32)],
    )(r_in, num, den, bc, nn)[0]


def _sc_logits(asv, adv, cvec, src_t, dst_t, w_t):
    mesh = plsc.VectorSubcoreMesh(core_axis_name="c", subcore_axis_name="s")
    cp = pltpu.CompilerParams()
    if "needs_layout_passes" in pltpu.CompilerParams.__dataclass_fields__:
        cp = dataclasses.replace(cp, needs_layout_passes=False)
    out_types = (
        jax.ShapeDtypeStruct((NUM_TILES, EPT_PAD), jnp.float32),
        jax.ShapeDtypeStruct((2, NP), jnp.float32),
    )

    @functools.partial(
        pl.kernel,
        out_type=out_types,
        mesh=mesh,
        compiler_params=cp,
        scratch_types=[
            pltpu.VMEM_SHARED((NP,), jnp.float32),
        ],
    )
    def k(as_hbm, ad_hbm, c_hbm, src_hbm, dst_hbm, w_hbm,
          pw_out, den_out, den_sh):
        cid = lax.axis_index("c")
        sid = lax.axis_index("s")
        tid = cid * 16 + sid

        def run(as_tab, ad_tab, c_tab, src_blk, dst_blk, w_blk, pwbuf,
                pbuf, zbuf, dsem):
            pltpu.sync_copy(as_hbm, as_tab)
            pltpu.sync_copy(ad_hbm, ad_tab)
            pltpu.sync_copy(c_hbm, c_tab)

            zv = jnp.zeros((16,), jnp.float32)

            @pl.loop(0, ROWS_PER_SUB // 16)
            def _(g):
                zbuf[pl.ds(g * 16, 16)] = zv

            base = sid * ROWS_PER_SUB
            pltpu.sync_copy(zbuf, den_sh.at[pl.ds(base, ROWS_PER_SUB)])
            plsc.subcore_barrier()

            c_vec = c_tab[...]

            @pl.loop(0, CHUNKS // 8)
            def _(blk):
                pltpu.sync_copy(src_hbm.at[tid, pl.ds(blk * 8, 8)], src_blk)
                pltpu.sync_copy(dst_hbm.at[tid, pl.ds(blk * 8, 8)], dst_blk)
                pltpu.sync_copy(w_hbm.at[tid, pl.ds(blk * 8, 8)], w_blk)

                dens = [None] * 8
                for jb in range(8):
                    for g in range(8):
                        sl = pl.ds(g * 16, 16)
                        fl = pl.ds(jb * CW + g * 16, 16)
                        src_v = src_blk[jb, sl]
                        dst_v = dst_blk[jb, sl]
                        as_v = plsc.load_gather(as_tab, [src_v])
                        ad_v = plsc.load_gather(ad_tab, [dst_v])
                        e = as_v + ad_v
                        e = jnp.maximum(e, NEG * e)
                        bb = c_vec + ad_v
                        bb = jnp.maximum(bb, NEG * bb)
                        p = jnp.exp(e - bb)
                        valid = jnp.where(
                            (blk * 8 + jb) * CW + g * 16 < EPT,
                            jnp.float32(1.0), jnp.float32(0.0))
                        p = p * valid
                        pwbuf[fl] = p * w_blk[jb, sl]
                        pbuf[fl] = p
                    dens[jb] = pltpu.async_copy(
                        pbuf.at[pl.ds(jb * CW, CW)],
                        den_sh.at[dst_blk.at[jb]], dsem, add=True)

                pltpu.sync_copy(
                    pwbuf, pw_out.at[tid, pl.ds(blk * (8 * CW), 8 * CW)])
                for jb in range(8):
                    dens[jb].wait()

            plsc.subcore_barrier()
            pltpu.sync_copy(den_sh.at[pl.ds(base, ROWS_PER_SUB)],
                            den_out.at[cid, pl.ds(base, ROWS_PER_SUB)])

        pl.run_scoped(
            run,
            as_tab=pltpu.VMEM((N,), jnp.float32),
            ad_tab=pltpu.VMEM((N,), jnp.float32),
            c_tab=pltpu.VMEM((16,), jnp.float32),
            src_blk=pltpu.VMEM((8, CW), jnp.int32),
            dst_blk=pltpu.VMEM((8, CW), jnp.int32),
            w_blk=pltpu.VMEM((8, CW), jnp.float32),
            pwbuf=pltpu.VMEM((8 * CW,), jnp.float32),
            pbuf=pltpu.VMEM((8 * CW,), jnp.float32),
            zbuf=pltpu.VMEM((ROWS_PER_SUB,), jnp.float32),
            dsem=pltpu.SemaphoreType.DMA,
        )

    return k(asv, adv, cvec, src_t, dst_t, w_t)


def _sc_rows(h2, pw, src_t, dst_t):
    mesh = plsc.VectorSubcoreMesh(core_axis_name="c", subcore_axis_name="s")
    cp = pltpu.CompilerParams()
    if "needs_layout_passes" in pltpu.CompilerParams.__dataclass_fields__:
        cp = dataclasses.replace(cp, needs_layout_passes=False)

    @functools.partial(
        pl.kernel,
        out_type=jax.ShapeDtypeStruct((2, NP, H), jnp.float32),
        mesh=mesh,
        compiler_params=cp,
        scratch_types=[
            pltpu.VMEM_SHARED((NP, H), jnp.float32),
        ],
    )
    def k(h2_hbm, pw_hbm, src_hbm, dst_hbm, num_out, num_sh):
        cid = lax.axis_index("c")
        sid = lax.axis_index("s")
        tid = cid * 16 + sid

        def run(src_blk, dst_blk, pw_blk, rows_a, rows_b, gsem_a, gsem_b,
                ssem_a, ssem_b):
            zv = jnp.zeros((16,), jnp.float32)

            @pl.loop(0, CW)
            def _(r):
                for k8 in range(8):
                    rows_a[r, pl.ds(k8 * 16, 16)] = zv

            base = sid * ROWS_PER_SUB
            for kk in range(5):
                pltpu.sync_copy(rows_a.at[pl.ds(0, CW)],
                                num_sh.at[pl.ds(base + kk * CW, CW)])
            plsc.subcore_barrier()

            bufs = (rows_a, rows_b)
            gsems = (gsem_a, gsem_b)
            ssems = (ssem_a, ssem_b)

            @pl.loop(0, CHUNKS // 8)
            def _(blk):
                pltpu.sync_copy(src_hbm.at[tid, pl.ds(blk * 8, 8)], src_blk)
                pltpu.sync_copy(dst_hbm.at[tid, pl.ds(blk * 8, 8)], dst_blk)
                pltpu.sync_copy(
                    pw_hbm.at[tid, pl.ds(blk * (8 * CW), 8 * CW)], pw_blk)

                gath = [None] * 8
                scat = [None] * 8
                for jb in range(2):
                    gath[jb] = pltpu.async_copy(
                        h2_hbm.at[src_blk.at[jb]], bufs[jb % 2],
                        gsems[jb % 2])

                for jb in range(8):
                    buf = bufs[jb % 2]
                    gath[jb].wait()
                    if jb >= 1:
                        scat[jb - 1].wait()
                        if jb < 7:
                            gath[jb + 1] = pltpu.async_copy(
                                h2_hbm.at[src_blk.at[jb + 1]],
                                bufs[(jb + 1) % 2], gsems[(jb + 1) % 2])

                    @plsc.parallel_loop(0, CW, unroll=4)
                    def _(r):
                        idxv = jnp.full((16,), jb * CW + r, jnp.int32)
                        sv = plsc.load_gather(pw_blk, [idxv])
                        for k8 in range(8):
                            slk = pl.ds(k8 * 16, 16)
                            buf[r, slk] = buf[r, slk] * sv

                    scat[jb] = pltpu.async_copy(
                        buf, num_sh.at[dst_blk.at[jb]], ssems[jb % 2],
                        add=True)

                scat[7].wait()

            plsc.subcore_barrier()
            pltpu.sync_copy(num_sh.at[pl.ds(base, ROWS_PER_SUB)],
                            num_out.at[cid, pl.ds(base, ROWS_PER_SUB)])

        pl.run_scoped(
            run,
            src_blk=pltpu.VMEM((8, CW), jnp.int32),
            dst_blk=pltpu.VMEM((8, CW), jnp.int32),
            pw_blk=pltpu.VMEM((8 * CW,), jnp.float32),
            rows_a=pltpu.VMEM((CW, H), jnp.float32),
            rows_b=pltpu.VMEM((CW, H), jnp.float32),
            gsem_a=pltpu.SemaphoreType.DMA,
            gsem_b=pltpu.SemaphoreType.DMA,
            ssem_a=pltpu.SemaphoreType.DMA,
            ssem_b=pltpu.SemaphoreType.DMA,
        )

    return k(h2, pw, src_t, dst_t)


def kernel(x, edge_index, edge_weight, numNode, W_lin, b_lin, W_conv,
           att_src, att_dst, b_conv):
    src = edge_index[0]
    dst = edge_index[1]
    pad = EPT_PAD - EPT
    src_t = jnp.pad(src.reshape(NUM_TILES, EPT), ((0, 0), (0, pad)))
    src_t = src_t.reshape(NUM_TILES, CHUNKS, CW)
    dst_t = jnp.pad(dst.reshape(NUM_TILES, EPT), ((0, 0), (0, pad)))
    dst_t = dst_t.reshape(NUM_TILES, CHUNKS, CW)
    w_t = jnp.pad(edge_weight.reshape(NUM_TILES, EPT), ((0, 0), (0, pad)))
    w_t = w_t.reshape(NUM_TILES, CHUNKS, CW)
    nn = jnp.asarray(numNode, jnp.int32).reshape(1)

    h, h2c, asv, adv, c = _first_tc(
        x, W_lin, b_lin.reshape(1, H), W_conv[0],
        att_src[0].reshape(1, H), att_dst[0].reshape(1, H))

    r = h
    for i in range(3):
        cvec = jnp.broadcast_to(c.reshape(()), (16,))
        pw, den = _sc_logits(asv.reshape(N), adv.reshape(N), cvec,
                             src_t, dst_t, w_t)
        num = _sc_rows(h2c, pw, src_t, dst_t)
        den = den.reshape(2, NP, 1)
        bc = b_conv[i].reshape(1, H)
        if i < 2:
            h, r, h2c, asv, adv, c = _mid_tc(
                h, r, num, den, bc, W_conv[i + 1],
                att_src[i + 1].reshape(1, H), att_dst[i + 1].reshape(1, H),
                nn, first=(i == 0))
        else:
            return _final_tc(r, num, den, bc, nn)
